# baseline (speedup 1.0000x reference)
import functools

import jax
import jax.numpy as jnp
from jax import lax
from jax.experimental import pallas as pl
from jax.experimental.pallas import tpu as pltpu
from jax.experimental.pallas import tpu_sc as plsc

B = 16
S = 4096
HALF = S // 2
LANES = 16
HCHUNKS = HALF // LANES
ACCS = 8


def _make_sc_kernel():
  mesh = plsc.VectorSubcoreMesh(core_axis_name="c", subcore_axis_name="s")

  @functools.partial(
      pl.kernel,
      mesh=mesh,
      out_type=jax.ShapeDtypeStruct((B, S), jnp.float32),
      scratch_types=[
          pltpu.VMEM((HALF,), jnp.float32),
          pltpu.VMEM((HALF,), jnp.float32),
          pltpu.SemaphoreType.DMA,
          pltpu.SemaphoreType.DMA,
      ],
      compiler_params=pltpu.CompilerParams(needs_layout_passes=False),
  )
  def cumsum_kernel(pad_hbm, out_hbm, buf_pre, buf, sem_pre, sem_own):
    cid = lax.axis_index("c")
    sid = lax.axis_index("s")
    wid = sid * 2 + cid
    row = wid // 2
    half = wid % 2
    own = half * HALF

    cp_pre = pltpu.async_copy(pad_hbm.at[row, pl.ds(0, HALF)], buf_pre,
                              sem_pre)
    cp_own = pltpu.async_copy(pad_hbm.at[row, pl.ds(own, HALF)], buf,
                              sem_own)
    cp_pre.wait()

    def acc_body(g, accs):
      return tuple(
          accs[k] + buf_pre[pl.ds((g * ACCS + k) * LANES, LANES)]
          for k in range(ACCS)
      )

    accs = lax.fori_loop(
        0, HCHUNKS // ACCS, acc_body,
        tuple(jnp.zeros((LANES,), jnp.float32) for _ in range(ACCS)))
    acc = accs[0]
    for k in range(1, ACCS):
      acc = acc + accs[k]
    prefix_total = jnp.sum(acc) * half.astype(jnp.float32)

    cp_own.wait()

    def scan_body(i, carry):
      base = i * LANES
      v = buf[pl.ds(base, LANES)]
      buf[pl.ds(base, LANES)] = plsc.cumsum(v) + carry
      return carry + jnp.sum(v)

    lax.fori_loop(0, HCHUNKS, scan_body, prefix_total - 1.0, unroll=16)

    pltpu.sync_copy(buf, out_hbm.at[row, pl.ds(own, HALF)])

  return cumsum_kernel


_sc_cumsum = _make_sc_kernel()


@jax.jit
def kernel(pad_masks):
  return _sc_cumsum(pad_masks)
